# baseline (speedup 1.0000x reference)
import jax
import jax.numpy as jnp
from jax import lax
from jax.experimental import pallas as pl
from jax.experimental.pallas import tpu as pltpu

_N, _C = 100000, 80
_CB = 8192
_NB = (_N + _CB - 1) // _CB


def _tc_body(xt_ref, lab_ref, sc_ref, out_ref):
    i = pl.program_id(0)
    x = xt_ref[...]
    lab = lab_ref[...]
    s = sc_ref[...]

    colg = i * _CB + lax.broadcasted_iota(jnp.int32, (1, _CB), 1)
    valid = colg < _N
    x = jnp.where(valid, x, 0.0)

    sig = 0.5 * jnp.tanh(0.5 * x) + 0.5
    sp = jnp.where(x > 12.0, x, -jnp.log(1.0 - sig))

    row = lax.broadcasted_iota(jnp.int32, x.shape, 0)
    m = row == lab
    sm = jnp.where(m, s, 0.0)
    d = sig - sm
    loss = (sp - x * sm) * d * d
    part = jnp.sum(jnp.where(valid, loss, 0.0))

    @pl.when(i == 0)
    def _init():
        out_ref[0, 0] = part

    @pl.when(i > 0)
    def _acc():
        out_ref[0, 0] += part


def kernel(pred, label, score):
    xt = pred.T
    lab2 = label.astype(jnp.int32).reshape(1, _N)
    sc2 = score.reshape(1, _N)
    total = pl.pallas_call(
        _tc_body,
        grid=(_NB,),
        in_specs=[
            pl.BlockSpec((_C, _CB), lambda i: (0, i)),
            pl.BlockSpec((1, _CB), lambda i: (0, i)),
            pl.BlockSpec((1, _CB), lambda i: (0, i)),
        ],
        out_specs=pl.BlockSpec(memory_space=pltpu.SMEM),
        out_shape=jax.ShapeDtypeStruct((1, 1), jnp.float32),
    )(xt, lab2, sc2)
    return total[0, 0] / _N


# speedup vs baseline: 1.0489x; 1.0489x over previous
import jax
import jax.numpy as jnp
from jax import lax
from jax.experimental import pallas as pl
from jax.experimental.pallas import tpu as pltpu

_N, _C = 100000, 80
_CB = 5120
_NB = (_N + _CB - 1) // _CB


def _tc_body(xt_ref, lab_ref, sc_ref, out_ref):
    i = pl.program_id(0)
    x = xt_ref[...]
    lab = lab_ref[...]
    s = sc_ref[...]

    colg = i * _CB + lax.broadcasted_iota(jnp.int32, (1, _CB), 1)
    valid = colg < _N
    x = jnp.where(valid, x, 0.0)

    sig = 0.5 * jnp.tanh(0.5 * x) + 0.5
    sp = jnp.where(x > 12.0, x, -jnp.log(1.0 - sig))

    row = lax.broadcasted_iota(jnp.int32, x.shape, 0)
    m = row == lab
    sm = jnp.where(m, s, 0.0)
    d = sig - sm
    loss = (sp - x * sm) * d * d
    part = jnp.sum(jnp.where(valid, loss, 0.0))

    @pl.when(i == 0)
    def _init():
        out_ref[0, 0] = part

    @pl.when(i > 0)
    def _acc():
        out_ref[0, 0] += part


def kernel(pred, label, score):
    xt = pred.T
    lab2 = label.astype(jnp.int32).reshape(1, _N)
    sc2 = score.reshape(1, _N)
    total = pl.pallas_call(
        _tc_body,
        grid=(_NB,),
        in_specs=[
            pl.BlockSpec((_C, _CB), lambda i: (0, i)),
            pl.BlockSpec((1, _CB), lambda i: (0, i)),
            pl.BlockSpec((1, _CB), lambda i: (0, i)),
        ],
        out_specs=pl.BlockSpec(memory_space=pltpu.SMEM),
        out_shape=jax.ShapeDtypeStruct((1, 1), jnp.float32),
    )(xt, lab2, sc2)
    return total[0, 0] / _N
